# Initial kernel scaffold; baseline (speedup 1.0000x reference)
import jax
import jax.numpy as jnp
from jax.experimental import pallas as pl


def kernel(x, edge_index, W_in, b_in, ln_in_g, ln_in_b, methylation, histones, log_depth, gcn_W, gcn_b, ln_g, ln_b, gate_W, gate_b, residual_weight, W_out, b_out, ln_out_g, ln_out_b):
    raise NotImplementedError("write your pallas kernel here")


# speedup vs baseline: 3.6999x; 3.6999x over previous
import functools

import jax
import jax.numpy as jnp
from jax import lax
from jax.experimental import pallas as pl
from jax.experimental.pallas import tpu as pltpu
from jax.experimental.pallas import tpu_sc as plsc

N = 10000
E = 320000
F = 128
NP = 10240
NC = 2
NS = 16
NW = NC * NS
CHUNK = 128
KCH = 80
EW = KCH * CHUNK
EPAD = NW * EW
RPS = NP // NS
RB = 1024
NBLK = NP // RB
MIN_DEPTH = 0.1
MAX_DEPTH = 3.0


def _segsum_body(t_hbm, src_hbm, dst_hbm, out_hbm, src_v, dst_v, buf, acc,
                 gsem):
    c = lax.axis_index("c")
    s = lax.axis_index("s")
    w = c * NS + s
    pltpu.sync_copy(src_hbm.at[w], src_v)
    pltpu.sync_copy(dst_hbm.at[w], dst_v)

    def _zb(i, carry):
        buf[i // 8, pl.ds((i % 8) * 16, 16)] = jnp.zeros((16,), jnp.float32)
        return carry

    lax.fori_loop(0, CHUNK * 8, _zb, 0)
    base = s * RPS
    for j in range(RPS // CHUNK):
        pltpu.sync_copy(buf, acc.at[pl.ds(base + j * CHUNK, CHUNK)])
    plsc.subcore_barrier()

    def _body(k, carry):
        pltpu.async_copy(t_hbm.at[src_v.at[k]], buf, gsem).wait()
        pltpu.sync_copy(buf, acc.at[dst_v.at[k]], add=True)
        return carry

    lax.fori_loop(0, KCH, _body, 0)
    plsc.subcore_barrier()
    pltpu.sync_copy(acc.at[pl.ds(base, RPS)], out_hbm.at[c, pl.ds(base, RPS)])


def _degree_body(dst_hbm, out_hbm, dst_v, buf, acc):
    c = lax.axis_index("c")
    s = lax.axis_index("s")
    w = c * NS + s
    pltpu.sync_copy(dst_hbm.at[w], dst_v)

    def _zb(i, carry):
        buf[i // 8, pl.ds((i % 8) * 16, 16)] = jnp.zeros((16,), jnp.float32)
        return carry

    lax.fori_loop(0, CHUNK * 8, _zb, 0)
    base = s * RPS
    for j in range(RPS // CHUNK):
        pltpu.sync_copy(buf, acc.at[pl.ds(base + j * CHUNK, CHUNK)])
    plsc.subcore_barrier()

    def _ones(i, carry):
        buf[i // 8, pl.ds((i % 8) * 16, 16)] = jnp.ones((16,), jnp.float32)
        return carry

    lax.fori_loop(0, CHUNK * 8, _ones, 0)

    def _body(k, carry):
        pltpu.sync_copy(buf, acc.at[dst_v.at[k]], add=True)
        return carry

    lax.fori_loop(0, KCH, _body, 0)
    plsc.subcore_barrier()
    pltpu.sync_copy(acc.at[pl.ds(base, RPS)], out_hbm.at[c, pl.ds(base, RPS)])


@functools.cache
def _sc_kernels():
    mesh = plsc.VectorSubcoreMesh(core_axis_name="c", subcore_axis_name="s",
                                  num_cores=NC, num_subcores=NS)
    segsum = pl.kernel(
        _segsum_body,
        out_type=jax.ShapeDtypeStruct((NC, NP, F), jnp.float32),
        mesh=mesh,
        scratch_types=[
            pltpu.VMEM((KCH, CHUNK), jnp.int32),
            pltpu.VMEM((KCH, CHUNK), jnp.int32),
            pltpu.VMEM((CHUNK, F), jnp.float32),
            pltpu.VMEM_SHARED((NP, F), jnp.float32),
            pltpu.SemaphoreType.DMA,
        ],
    )
    degree = pl.kernel(
        _degree_body,
        out_type=jax.ShapeDtypeStruct((NC, NP, F), jnp.float32),
        mesh=mesh,
        scratch_types=[
            pltpu.VMEM((KCH, CHUNK), jnp.int32),
            pltpu.VMEM((CHUNK, F), jnp.float32),
            pltpu.VMEM_SHARED((NP, F), jnp.float32),
        ],
    )
    return segsum, degree


def _segsum(tp, src, dst):
    return _sc_kernels()[0](tp, src, dst)


def _degree(dst):
    return _sc_kernels()[1](dst)


def _ln(x, g, b):
    m = jnp.mean(x, axis=-1, keepdims=True)
    v = jnp.mean((x - m) ** 2, axis=-1, keepdims=True)
    return (x - m) * lax.rsqrt(v + 1e-5) * g + b


def _mT(x, w):
    return lax.dot_general(x, w, (((1,), (1,)), ((), ())),
                           preferred_element_type=jnp.float32)


_row = pl.BlockSpec((RB, F), lambda i: (i, 0))
_par = pl.BlockSpec((1, F), lambda i: (0, 0))
_wgt = pl.BlockSpec((F, F), lambda i: (0, 0))


def _kdis_body(d0_ref, d1_ref, o_ref):
    dg = d0_ref[...] + d1_ref[...]
    o_ref[...] = jnp.where(dg > 0.0, lax.rsqrt(jnp.maximum(dg, 1.0)), 0.0)


_kdis = pl.pallas_call(
    _kdis_body,
    grid=(1,),
    in_specs=[pl.BlockSpec((80, 128), lambda i: (0, 0))] * 2,
    out_specs=pl.BlockSpec((80, 128), lambda i: (0, 0)),
    out_shape=jax.ShapeDtypeStruct((80, 128), jnp.float32),
)


def _k0_body(x_ref, win_ref, bin_ref, g_ref, b_ref, epi_ref, w0_ref, dis_ref,
             h_ref, t0_ref):
    h = _mT(x_ref[...], win_ref[...]) + bin_ref[...]
    h = _ln(h, g_ref[...], b_ref[...])
    h = jnp.maximum(h, 0.0) * epi_ref[...]
    h_ref[...] = h
    t0_ref[...] = _mT(h, w0_ref[...]) * dis_ref[...]


_k0 = pl.pallas_call(
    _k0_body,
    grid=(NBLK,),
    in_specs=[_row, _wgt, _par, _par, _par, _par, _wgt, _row],
    out_specs=[_row, _row],
    out_shape=[jax.ShapeDtypeStruct((NP, F), jnp.float32)] * 2,
)


def _k2_body(s0_ref, s1_ref, dis_ref, b_ref, g_ref, be_ref, w1_ref,
             cur_ref, t1_ref):
    dis = dis_ref[...]
    agg = dis * (s0_ref[...] + s1_ref[...]) + b_ref[...]
    cur = _ln(agg, g_ref[...], be_ref[...])
    cur_ref[...] = cur
    t1_ref[...] = _mT(cur, w1_ref[...]) * dis


_k2 = pl.pallas_call(
    _k2_body,
    grid=(NBLK,),
    in_specs=[_row, _row, _row, _par, _par, _par, _wgt],
    out_specs=[_row, _row],
    out_shape=[jax.ShapeDtypeStruct((NP, F), jnp.float32)] * 2,
)


def _k3_body(s0_ref, s1_ref, dis_ref, b_ref, g_ref, be_ref, cur_ref,
             gwa_ref, gwb_ref, gb_ref, wn_ref, curo_ref, tn_ref):
    dis = dis_ref[...]
    cur = cur_ref[...]
    h_new = _ln(dis * (s0_ref[...] + s1_ref[...]) + b_ref[...],
                g_ref[...], be_ref[...])
    gate = jax.nn.sigmoid(_mT(cur, gwa_ref[...]) + _mT(h_new, gwb_ref[...])
                          + gb_ref[...])
    curo = gate * h_new + (1.0 - gate) * cur
    curo_ref[...] = curo
    tn_ref[...] = _mT(curo, wn_ref[...]) * dis


_k3 = pl.pallas_call(
    _k3_body,
    grid=(NBLK,),
    in_specs=[_row, _row, _row, _par, _par, _par, _row, _wgt, _wgt, _par,
              _wgt],
    out_specs=[_row, _row],
    out_shape=[jax.ShapeDtypeStruct((NP, F), jnp.float32)] * 2,
)


def _k4_body(s0_ref, s1_ref, dis_ref, b_ref, g_ref, be_ref, cur_ref,
             gwa_ref, gwb_ref, gb_ref, hh_ref, rw_ref, h_ref, cn_ref,
             w0_ref, k_ref, hhn_ref, t0n_ref):
    dis = dis_ref[...]
    cur = cur_ref[...]
    h_new = _ln(dis * (s0_ref[...] + s1_ref[...]) + b_ref[...],
                g_ref[...], be_ref[...])
    gate = jax.nn.sigmoid(_mT(cur, gwa_ref[...]) + _mT(h_new, gwb_ref[...])
                          + gb_ref[...])
    curo = gate * h_new + (1.0 - gate) * cur
    kk = jnp.tanh(curo) + rw_ref[...] * hh_ref[...]
    k_ref[...] = kk
    hhn = h_ref[...] + cn_ref[...] * kk
    hhn_ref[...] = hhn
    t0n_ref[...] = _mT(hhn, w0_ref[...]) * dis


_k4 = pl.pallas_call(
    _k4_body,
    grid=(NBLK,),
    in_specs=[_row, _row, _row, _par, _par, _par, _row, _wgt, _wgt, _par,
              _row, _par, _row, _par, _wgt],
    out_specs=[_row, _row, _row],
    out_shape=[jax.ShapeDtypeStruct((NP, F), jnp.float32)] * 3,
)


def _k4f_body(s0_ref, s1_ref, dis_ref, b_ref, g_ref, be_ref, cur_ref,
              gwa_ref, gwb_ref, gb_ref, hh_ref, rw_ref, k_ref):
    dis = dis_ref[...]
    cur = cur_ref[...]
    h_new = _ln(dis * (s0_ref[...] + s1_ref[...]) + b_ref[...],
                g_ref[...], be_ref[...])
    gate = jax.nn.sigmoid(_mT(cur, gwa_ref[...]) + _mT(h_new, gwb_ref[...])
                          + gb_ref[...])
    curo = gate * h_new + (1.0 - gate) * cur
    k_ref[...] = jnp.tanh(curo) + rw_ref[...] * hh_ref[...]


_k4f = pl.pallas_call(
    _k4f_body,
    grid=(NBLK,),
    in_specs=[_row, _row, _row, _par, _par, _par, _row, _wgt, _wgt, _par,
              _row, _par],
    out_specs=_row,
    out_shape=jax.ShapeDtypeStruct((NP, F), jnp.float32),
)


def _kfin_body(h_ref, k1_ref, k2_ref, k3_ref, k4_ref, dt6_ref, wo_ref,
               bo_ref, go_ref, beo_ref, o_ref):
    i = pl.program_id(0)
    h2 = h_ref[...] + dt6_ref[...] * (k1_ref[...] + 2.0 * k2_ref[...]
                                      + 2.0 * k3_ref[...] + k4_ref[...])
    o = _ln(_mT(h2, wo_ref[...]) + bo_ref[...], go_ref[...], beo_ref[...])
    rows = lax.broadcasted_iota(jnp.int32, (RB, 1), 0) + i * RB
    o = jnp.where(rows < N, o, 0.0)
    part = jnp.sum(o, axis=0, keepdims=True) * (1.0 / N)

    @pl.when(i == 0)
    def _():
        o_ref[...] = jnp.zeros_like(o_ref)

    o_ref[...] += part


_kfin = pl.pallas_call(
    _kfin_body,
    grid=(NBLK,),
    in_specs=[_row, _row, _row, _row, _row, _par, _wgt, _par, _par, _par],
    out_specs=pl.BlockSpec((1, F), lambda i: (0, 0)),
    out_shape=jax.ShapeDtypeStruct((1, F), jnp.float32),
)


def kernel(x, edge_index, W_in, b_in, ln_in_g, ln_in_b, methylation, histones,
           log_depth, gcn_W, gcn_b, ln_g, ln_b, gate_W, gate_b,
           residual_weight, W_out, b_out, ln_out_g, ln_out_b):
    f32 = jnp.float32
    xp = jnp.pad(x, ((0, NP - N), (0, 0)))
    src = jnp.pad(edge_index[0], (0, EPAD - E),
                  constant_values=NP - 1).reshape(NW, KCH, CHUNK)
    dst = jnp.pad(edge_index[1], (0, EPAD - E),
                  constant_values=NP - 1).reshape(NW, KCH, CHUNK)

    deg2 = _degree(dst)
    d0 = deg2[0, :, 0].reshape(80, 128)
    d1 = deg2[1, :, 0].reshape(80, 128)
    dis2d = _kdis(d0, d1)
    dis_bc = jnp.broadcast_to(dis2d.reshape(NP, 1), (NP, F))

    def b2(v):
        return v.reshape(1, F)

    def full(v):
        return jnp.full((1, F), v, dtype=f32)

    ms = jnp.mean(jax.nn.sigmoid(methylation))
    hm = jax.nn.sigmoid(histones)
    activation_marks = (hm[0] + hm[2]) / 2.0
    repression_marks = (hm[1] + hm[3]) / 2.0
    acc_s = jnp.clip(activation_marks - repression_marks + 0.5, 0.0, 1.0)
    epi = acc_s * (1.0 - ms)
    depth = lax.stop_gradient(jnp.clip(jnp.exp(log_depth), MIN_DEPTH,
                                       MAX_DEPTH))
    dt = depth

    gwa = gate_W[:, :F]
    gwb = gate_W[:, F:]
    gb = b2(gate_b)
    rw = full(residual_weight)

    h, t = _k0(xp, W_in, b2(b_in), b2(ln_in_g), b2(ln_in_b), full(epi),
               gcn_W[0], dis_bc)

    def seg(tp):
        S = _segsum(tp, src, dst)
        return S[0], S[1]

    hh = h
    ks = []
    for it in range(4):
        s0, s1 = seg(t)
        cur1, t1 = _k2(s0, s1, dis_bc, b2(gcn_b[0]), b2(ln_g[0]),
                       b2(ln_b[0]), gcn_W[1])
        s0, s1 = seg(t1)
        cur2, t2 = _k3(s0, s1, dis_bc, b2(gcn_b[1]), b2(ln_g[1]),
                       b2(ln_b[1]), cur1, gwa, gwb, gb, gcn_W[2])
        s0, s1 = seg(t2)
        if it < 3:
            cnext = dt * (0.5 if it < 2 else 1.0)
            kk, hh, t = _k4(s0, s1, dis_bc, b2(gcn_b[2]), b2(ln_g[2]),
                            b2(ln_b[2]), cur2, gwa, gwb, gb, hh, rw, h,
                            full(cnext), gcn_W[0])
        else:
            kk = _k4f(s0, s1, dis_bc, b2(gcn_b[2]), b2(ln_g[2]),
                      b2(ln_b[2]), cur2, gwa, gwb, gb, hh, rw)
        ks.append(kk)

    out = _kfin(h, ks[0], ks[1], ks[2], ks[3], full(dt / 6.0), W_out,
                b2(b_out), b2(ln_out_g), b2(ln_out_b))
    return out


# speedup vs baseline: 4.0154x; 1.0853x over previous
import functools

import jax
import jax.numpy as jnp
from jax import lax
from jax.experimental import pallas as pl
from jax.experimental.pallas import tpu as pltpu
from jax.experimental.pallas import tpu_sc as plsc

N = 10000
E = 320000
F = 128
NP = 10240
NC = 2
NS = 16
NW = NC * NS
CHUNK = 128
KCH = 80
EW = KCH * CHUNK
EPAD = NW * EW
RPS = NP // NS
RB = 1024
NBLK = NP // RB
MIN_DEPTH = 0.1
MAX_DEPTH = 3.0


G = 8
NG = KCH // G


def _segsum_body(t_hbm, src_hbm, dst_hbm, out_hbm, srcr, dstr, bufs, acc,
                 gsem, isem, ssem):
    c = lax.axis_index("c")
    s = lax.axis_index("s")
    w = c * NS + s
    pltpu.sync_copy(src_hbm.at[w, pl.ds(0, G)], srcr.at[0])
    pltpu.sync_copy(dst_hbm.at[w, pl.ds(0, G)], dstr.at[0])

    def _zb(i, carry):
        bufs[0, i // 8, pl.ds((i % 8) * 16, 16)] = jnp.zeros((16,),
                                                             jnp.float32)
        return carry

    lax.fori_loop(0, CHUNK * 8, _zb, 0)
    base = s * RPS
    for j in range(RPS // CHUNK):
        pltpu.sync_copy(bufs.at[0], acc.at[pl.ds(base + j * CHUNK, CHUNK)])
    plsc.subcore_barrier()

    def _wait_chunk(sem, bslot):
        pltpu.make_async_copy(t_hbm.at[pl.ds(0, CHUNK)], bufs.at[bslot],
                              sem).wait()

    pltpu.async_copy(t_hbm.at[srcr.at[0, 0]], bufs.at[0], gsem)

    def _grp(g, carry):
        slot = g % 2
        nslot = 1 - slot

        @pl.when(g + 1 < NG)
        def _():
            pltpu.async_copy(src_hbm.at[w, pl.ds((g + 1) * G, G)],
                             srcr.at[nslot], isem)
            pltpu.async_copy(dst_hbm.at[w, pl.ds((g + 1) * G, G)],
                             dstr.at[nslot], isem)

        for j in range(G):
            b = j % 2
            _wait_chunk(gsem, b)
            if j == 0:
                @pl.when(g > 0)
                def _():
                    _wait_chunk(ssem, 1 - b)
            else:
                _wait_chunk(ssem, 1 - b)
            if j + 1 < G:
                pltpu.async_copy(t_hbm.at[srcr.at[slot, j + 1]],
                                 bufs.at[1 - b], gsem)
            else:
                @pl.when(g + 1 < NG)
                def _():
                    pltpu.make_async_copy(src_hbm.at[w, pl.ds(0, G)],
                                          srcr.at[nslot], isem).wait()
                    pltpu.make_async_copy(dst_hbm.at[w, pl.ds(0, G)],
                                          dstr.at[nslot], isem).wait()
                    pltpu.async_copy(t_hbm.at[srcr.at[nslot, 0]],
                                     bufs.at[1 - b], gsem)
            pltpu.async_copy(bufs.at[b], acc.at[dstr.at[slot, j]], ssem,
                             add=True)
        return carry

    lax.fori_loop(0, NG, _grp, 0)
    _wait_chunk(ssem, 1)
    plsc.subcore_barrier()
    pltpu.sync_copy(acc.at[pl.ds(base, RPS)], out_hbm.at[c, pl.ds(base, RPS)])


def _degree_body(dst_hbm, out_hbm, dst_v, buf, acc):
    c = lax.axis_index("c")
    s = lax.axis_index("s")
    w = c * NS + s
    pltpu.sync_copy(dst_hbm.at[w], dst_v)

    def _zb(i, carry):
        buf[i // 8, pl.ds((i % 8) * 16, 16)] = jnp.zeros((16,), jnp.float32)
        return carry

    lax.fori_loop(0, CHUNK * 8, _zb, 0)
    base = s * RPS
    for j in range(RPS // CHUNK):
        pltpu.sync_copy(buf, acc.at[pl.ds(base + j * CHUNK, CHUNK)])
    plsc.subcore_barrier()

    def _ones(i, carry):
        buf[i // 8, pl.ds((i % 8) * 16, 16)] = jnp.ones((16,), jnp.float32)
        return carry

    lax.fori_loop(0, CHUNK * 8, _ones, 0)

    def _body(k, carry):
        pltpu.sync_copy(buf, acc.at[dst_v.at[k]], add=True)
        return carry

    lax.fori_loop(0, KCH, _body, 0)
    plsc.subcore_barrier()
    pltpu.sync_copy(acc.at[pl.ds(base, RPS)], out_hbm.at[c, pl.ds(base, RPS)])


@functools.cache
def _sc_kernels():
    mesh = plsc.VectorSubcoreMesh(core_axis_name="c", subcore_axis_name="s",
                                  num_cores=NC, num_subcores=NS)
    segsum = pl.kernel(
        _segsum_body,
        out_type=jax.ShapeDtypeStruct((NC, NP, F), jnp.float32),
        mesh=mesh,
        scratch_types=[
            pltpu.VMEM((2, G, CHUNK), jnp.int32),
            pltpu.VMEM((2, G, CHUNK), jnp.int32),
            pltpu.VMEM((2, CHUNK, F), jnp.float32),
            pltpu.VMEM_SHARED((NP, F), jnp.float32),
            pltpu.SemaphoreType.DMA,
            pltpu.SemaphoreType.DMA,
            pltpu.SemaphoreType.DMA,
        ],
    )
    degree = pl.kernel(
        _degree_body,
        out_type=jax.ShapeDtypeStruct((NC, NP, F), jnp.float32),
        mesh=mesh,
        scratch_types=[
            pltpu.VMEM((KCH, CHUNK), jnp.int32),
            pltpu.VMEM((CHUNK, F), jnp.float32),
            pltpu.VMEM_SHARED((NP, F), jnp.float32),
        ],
    )
    return segsum, degree


def _segsum(tp, src, dst):
    return _sc_kernels()[0](tp, src, dst)


def _degree(dst):
    return _sc_kernels()[1](dst)


def _ln(x, g, b):
    m = jnp.mean(x, axis=-1, keepdims=True)
    v = jnp.mean((x - m) ** 2, axis=-1, keepdims=True)
    return (x - m) * lax.rsqrt(v + 1e-5) * g + b


def _mT(x, w):
    return lax.dot_general(x, w, (((1,), (1,)), ((), ())),
                           preferred_element_type=jnp.float32)


_row = pl.BlockSpec((RB, F), lambda i: (i, 0))
_par = pl.BlockSpec((1, F), lambda i: (0, 0))
_wgt = pl.BlockSpec((F, F), lambda i: (0, 0))


def _kdis_body(d0_ref, d1_ref, o_ref):
    dg = d0_ref[...] + d1_ref[...]
    o_ref[...] = jnp.where(dg > 0.0, lax.rsqrt(jnp.maximum(dg, 1.0)), 0.0)


_kdis = pl.pallas_call(
    _kdis_body,
    grid=(1,),
    in_specs=[pl.BlockSpec((80, 128), lambda i: (0, 0))] * 2,
    out_specs=pl.BlockSpec((80, 128), lambda i: (0, 0)),
    out_shape=jax.ShapeDtypeStruct((80, 128), jnp.float32),
)


def _k0_body(x_ref, win_ref, bin_ref, g_ref, b_ref, epi_ref, w0_ref, dis_ref,
             h_ref, t0_ref):
    h = _mT(x_ref[...], win_ref[...]) + bin_ref[...]
    h = _ln(h, g_ref[...], b_ref[...])
    h = jnp.maximum(h, 0.0) * epi_ref[...]
    h_ref[...] = h
    t0_ref[...] = _mT(h, w0_ref[...]) * dis_ref[...]


_k0 = pl.pallas_call(
    _k0_body,
    grid=(NBLK,),
    in_specs=[_row, _wgt, _par, _par, _par, _par, _wgt, _row],
    out_specs=[_row, _row],
    out_shape=[jax.ShapeDtypeStruct((NP, F), jnp.float32)] * 2,
)


def _k2_body(s0_ref, s1_ref, dis_ref, b_ref, g_ref, be_ref, w1_ref,
             cur_ref, t1_ref):
    dis = dis_ref[...]
    agg = dis * (s0_ref[...] + s1_ref[...]) + b_ref[...]
    cur = _ln(agg, g_ref[...], be_ref[...])
    cur_ref[...] = cur
    t1_ref[...] = _mT(cur, w1_ref[...]) * dis


_k2 = pl.pallas_call(
    _k2_body,
    grid=(NBLK,),
    in_specs=[_row, _row, _row, _par, _par, _par, _wgt],
    out_specs=[_row, _row],
    out_shape=[jax.ShapeDtypeStruct((NP, F), jnp.float32)] * 2,
)


def _k3_body(s0_ref, s1_ref, dis_ref, b_ref, g_ref, be_ref, cur_ref,
             gwa_ref, gwb_ref, gb_ref, wn_ref, curo_ref, tn_ref):
    dis = dis_ref[...]
    cur = cur_ref[...]
    h_new = _ln(dis * (s0_ref[...] + s1_ref[...]) + b_ref[...],
                g_ref[...], be_ref[...])
    gate = jax.nn.sigmoid(_mT(cur, gwa_ref[...]) + _mT(h_new, gwb_ref[...])
                          + gb_ref[...])
    curo = gate * h_new + (1.0 - gate) * cur
    curo_ref[...] = curo
    tn_ref[...] = _mT(curo, wn_ref[...]) * dis


_k3 = pl.pallas_call(
    _k3_body,
    grid=(NBLK,),
    in_specs=[_row, _row, _row, _par, _par, _par, _row, _wgt, _wgt, _par,
              _wgt],
    out_specs=[_row, _row],
    out_shape=[jax.ShapeDtypeStruct((NP, F), jnp.float32)] * 2,
)


def _k4_body(s0_ref, s1_ref, dis_ref, b_ref, g_ref, be_ref, cur_ref,
             gwa_ref, gwb_ref, gb_ref, hh_ref, rw_ref, h_ref, cn_ref,
             w0_ref, k_ref, hhn_ref, t0n_ref):
    dis = dis_ref[...]
    cur = cur_ref[...]
    h_new = _ln(dis * (s0_ref[...] + s1_ref[...]) + b_ref[...],
                g_ref[...], be_ref[...])
    gate = jax.nn.sigmoid(_mT(cur, gwa_ref[...]) + _mT(h_new, gwb_ref[...])
                          + gb_ref[...])
    curo = gate * h_new + (1.0 - gate) * cur
    kk = jnp.tanh(curo) + rw_ref[...] * hh_ref[...]
    k_ref[...] = kk
    hhn = h_ref[...] + cn_ref[...] * kk
    hhn_ref[...] = hhn
    t0n_ref[...] = _mT(hhn, w0_ref[...]) * dis


_k4 = pl.pallas_call(
    _k4_body,
    grid=(NBLK,),
    in_specs=[_row, _row, _row, _par, _par, _par, _row, _wgt, _wgt, _par,
              _row, _par, _row, _par, _wgt],
    out_specs=[_row, _row, _row],
    out_shape=[jax.ShapeDtypeStruct((NP, F), jnp.float32)] * 3,
)


def _k4f_body(s0_ref, s1_ref, dis_ref, b_ref, g_ref, be_ref, cur_ref,
              gwa_ref, gwb_ref, gb_ref, hh_ref, rw_ref, k_ref):
    dis = dis_ref[...]
    cur = cur_ref[...]
    h_new = _ln(dis * (s0_ref[...] + s1_ref[...]) + b_ref[...],
                g_ref[...], be_ref[...])
    gate = jax.nn.sigmoid(_mT(cur, gwa_ref[...]) + _mT(h_new, gwb_ref[...])
                          + gb_ref[...])
    curo = gate * h_new + (1.0 - gate) * cur
    k_ref[...] = jnp.tanh(curo) + rw_ref[...] * hh_ref[...]


_k4f = pl.pallas_call(
    _k4f_body,
    grid=(NBLK,),
    in_specs=[_row, _row, _row, _par, _par, _par, _row, _wgt, _wgt, _par,
              _row, _par],
    out_specs=_row,
    out_shape=jax.ShapeDtypeStruct((NP, F), jnp.float32),
)


def _kfin_body(h_ref, k1_ref, k2_ref, k3_ref, k4_ref, dt6_ref, wo_ref,
               bo_ref, go_ref, beo_ref, o_ref):
    i = pl.program_id(0)
    h2 = h_ref[...] + dt6_ref[...] * (k1_ref[...] + 2.0 * k2_ref[...]
                                      + 2.0 * k3_ref[...] + k4_ref[...])
    o = _ln(_mT(h2, wo_ref[...]) + bo_ref[...], go_ref[...], beo_ref[...])
    rows = lax.broadcasted_iota(jnp.int32, (RB, 1), 0) + i * RB
    o = jnp.where(rows < N, o, 0.0)
    part = jnp.sum(o, axis=0, keepdims=True) * (1.0 / N)

    @pl.when(i == 0)
    def _():
        o_ref[...] = jnp.zeros_like(o_ref)

    o_ref[...] += part


_kfin = pl.pallas_call(
    _kfin_body,
    grid=(NBLK,),
    in_specs=[_row, _row, _row, _row, _row, _par, _wgt, _par, _par, _par],
    out_specs=pl.BlockSpec((1, F), lambda i: (0, 0)),
    out_shape=jax.ShapeDtypeStruct((1, F), jnp.float32),
)


def kernel(x, edge_index, W_in, b_in, ln_in_g, ln_in_b, methylation, histones,
           log_depth, gcn_W, gcn_b, ln_g, ln_b, gate_W, gate_b,
           residual_weight, W_out, b_out, ln_out_g, ln_out_b):
    f32 = jnp.float32
    xp = jnp.pad(x, ((0, NP - N), (0, 0)))
    src = jnp.pad(edge_index[0], (0, EPAD - E),
                  constant_values=NP - 1).reshape(NW, KCH, CHUNK)
    dst = jnp.pad(edge_index[1], (0, EPAD - E),
                  constant_values=NP - 1).reshape(NW, KCH, CHUNK)

    deg2 = _degree(dst)
    d0 = deg2[0, :, 0].reshape(80, 128)
    d1 = deg2[1, :, 0].reshape(80, 128)
    dis2d = _kdis(d0, d1)
    dis_bc = jnp.broadcast_to(dis2d.reshape(NP, 1), (NP, F))

    def b2(v):
        return v.reshape(1, F)

    def full(v):
        return jnp.full((1, F), v, dtype=f32)

    ms = jnp.mean(jax.nn.sigmoid(methylation))
    hm = jax.nn.sigmoid(histones)
    activation_marks = (hm[0] + hm[2]) / 2.0
    repression_marks = (hm[1] + hm[3]) / 2.0
    acc_s = jnp.clip(activation_marks - repression_marks + 0.5, 0.0, 1.0)
    epi = acc_s * (1.0 - ms)
    depth = lax.stop_gradient(jnp.clip(jnp.exp(log_depth), MIN_DEPTH,
                                       MAX_DEPTH))
    dt = depth

    gwa = gate_W[:, :F]
    gwb = gate_W[:, F:]
    gb = b2(gate_b)
    rw = full(residual_weight)

    h, t = _k0(xp, W_in, b2(b_in), b2(ln_in_g), b2(ln_in_b), full(epi),
               gcn_W[0], dis_bc)

    def seg(tp):
        S = _segsum(tp, src, dst)
        return S[0], S[1]

    hh = h
    ks = []
    for it in range(4):
        s0, s1 = seg(t)
        cur1, t1 = _k2(s0, s1, dis_bc, b2(gcn_b[0]), b2(ln_g[0]),
                       b2(ln_b[0]), gcn_W[1])
        s0, s1 = seg(t1)
        cur2, t2 = _k3(s0, s1, dis_bc, b2(gcn_b[1]), b2(ln_g[1]),
                       b2(ln_b[1]), cur1, gwa, gwb, gb, gcn_W[2])
        s0, s1 = seg(t2)
        if it < 3:
            cnext = dt * (0.5 if it < 2 else 1.0)
            kk, hh, t = _k4(s0, s1, dis_bc, b2(gcn_b[2]), b2(ln_g[2]),
                            b2(ln_b[2]), cur2, gwa, gwb, gb, hh, rw, h,
                            full(cnext), gcn_W[0])
        else:
            kk = _k4f(s0, s1, dis_bc, b2(gcn_b[2]), b2(ln_g[2]),
                      b2(ln_b[2]), cur2, gwa, gwb, gb, hh, rw)
        ks.append(kk)

    out = _kfin(h, ks[0], ks[1], ks[2], ks[3], full(dt / 6.0), W_out,
                b2(b_out), b2(ln_out_g), b2(ln_out_b))
    return out


# speedup vs baseline: 4.0952x; 1.0199x over previous
import functools

import jax
import jax.numpy as jnp
from jax import lax
from jax.experimental import pallas as pl
from jax.experimental.pallas import tpu as pltpu
from jax.experimental.pallas import tpu_sc as plsc

N = 10000
E = 320000
F = 128
NP = 10240
NC = 2
NS = 16
NW = NC * NS
CHUNK = 64
KCH = 160
EW = KCH * CHUNK
EPAD = NW * EW
RPS = NP // NS
RB = 1024
NBLK = NP // RB
MIN_DEPTH = 0.1
MAX_DEPTH = 3.0


G = 8
NG = KCH // G
NBUF = 4
PIPE = 3


def _segsum_body(t_hbm, src_hbm, dst_hbm, out_hbm, srcr, dstr, bufs, acc,
                 gsem, isem, ssem):
    c = lax.axis_index("c")
    s = lax.axis_index("s")
    w = c * NS + s
    pltpu.sync_copy(src_hbm.at[w, pl.ds(0, G)], srcr.at[0])
    pltpu.sync_copy(dst_hbm.at[w, pl.ds(0, G)], dstr.at[0])

    def _zb(i, carry):
        bufs[0, i // 8, pl.ds((i % 8) * 16, 16)] = jnp.zeros((16,),
                                                             jnp.float32)
        return carry

    lax.fori_loop(0, CHUNK * 8, _zb, 0)
    base = s * RPS
    for j in range(RPS // CHUNK):
        pltpu.sync_copy(bufs.at[0], acc.at[pl.ds(base + j * CHUNK, CHUNK)])
    plsc.subcore_barrier()

    def _wait_chunk(sem, bslot):
        pltpu.make_async_copy(t_hbm.at[pl.ds(0, CHUNK)], bufs.at[bslot],
                              sem).wait()

    for j in range(PIPE):
        pltpu.async_copy(t_hbm.at[srcr.at[0, j]], bufs.at[j], gsem)

    def _grp(g, carry):
        slot = g % 2
        nslot = 1 - slot

        @pl.when(g + 1 < NG)
        def _():
            pltpu.async_copy(src_hbm.at[w, pl.ds((g + 1) * G, G)],
                             srcr.at[nslot], isem)
            pltpu.async_copy(dst_hbm.at[w, pl.ds((g + 1) * G, G)],
                             dstr.at[nslot], isem)

        for j in range(G):
            b = j % NBUF
            _wait_chunk(gsem, b)
            if j == 0:
                @pl.when(g > 0)
                def _():
                    _wait_chunk(ssem, (j - 1) % NBUF)
            else:
                _wait_chunk(ssem, (j - 1) % NBUF)
            if j + PIPE < G:
                pltpu.async_copy(t_hbm.at[srcr.at[slot, j + PIPE]],
                                 bufs.at[(j + PIPE) % NBUF], gsem)
            else:
                if j + PIPE == G:
                    @pl.when(g + 1 < NG)
                    def _():
                        pltpu.make_async_copy(src_hbm.at[w, pl.ds(0, G)],
                                              srcr.at[nslot], isem).wait()
                        pltpu.make_async_copy(dst_hbm.at[w, pl.ds(0, G)],
                                              dstr.at[nslot], isem).wait()

                @pl.when(g + 1 < NG)
                def _():
                    pltpu.async_copy(t_hbm.at[srcr.at[nslot, j + PIPE - G]],
                                     bufs.at[(j + PIPE) % NBUF], gsem)
            pltpu.async_copy(bufs.at[b], acc.at[dstr.at[slot, j]], ssem,
                             add=True)
        return carry

    lax.fori_loop(0, NG, _grp, 0)
    _wait_chunk(ssem, (KCH - 1) % NBUF)
    plsc.subcore_barrier()
    pltpu.sync_copy(acc.at[pl.ds(base, RPS)], out_hbm.at[c, pl.ds(base, RPS)])


def _degree_body(dst_hbm, out_hbm, dst_v, buf, acc):
    c = lax.axis_index("c")
    s = lax.axis_index("s")
    w = c * NS + s
    pltpu.sync_copy(dst_hbm.at[w], dst_v)

    def _zb(i, carry):
        buf[i // 8, pl.ds((i % 8) * 16, 16)] = jnp.zeros((16,), jnp.float32)
        return carry

    lax.fori_loop(0, CHUNK * 8, _zb, 0)
    base = s * RPS
    for j in range(RPS // CHUNK):
        pltpu.sync_copy(buf, acc.at[pl.ds(base + j * CHUNK, CHUNK)])
    plsc.subcore_barrier()

    def _ones(i, carry):
        buf[i // 8, pl.ds((i % 8) * 16, 16)] = jnp.ones((16,), jnp.float32)
        return carry

    lax.fori_loop(0, CHUNK * 8, _ones, 0)

    def _body(k, carry):
        pltpu.sync_copy(buf, acc.at[dst_v.at[k]], add=True)
        return carry

    lax.fori_loop(0, KCH, _body, 0)
    plsc.subcore_barrier()
    pltpu.sync_copy(acc.at[pl.ds(base, RPS)], out_hbm.at[c, pl.ds(base, RPS)])


@functools.cache
def _sc_kernels():
    mesh = plsc.VectorSubcoreMesh(core_axis_name="c", subcore_axis_name="s",
                                  num_cores=NC, num_subcores=NS)
    segsum = pl.kernel(
        _segsum_body,
        out_type=jax.ShapeDtypeStruct((NC, NP, F), jnp.float32),
        mesh=mesh,
        scratch_types=[
            pltpu.VMEM((2, G, CHUNK), jnp.int32),
            pltpu.VMEM((2, G, CHUNK), jnp.int32),
            pltpu.VMEM((NBUF, CHUNK, F), jnp.float32),
            pltpu.VMEM_SHARED((NP, F), jnp.float32),
            pltpu.SemaphoreType.DMA,
            pltpu.SemaphoreType.DMA,
            pltpu.SemaphoreType.DMA,
        ],
    )
    degree = pl.kernel(
        _degree_body,
        out_type=jax.ShapeDtypeStruct((NC, NP, F), jnp.float32),
        mesh=mesh,
        scratch_types=[
            pltpu.VMEM((KCH, CHUNK), jnp.int32),
            pltpu.VMEM((CHUNK, F), jnp.float32),
            pltpu.VMEM_SHARED((NP, F), jnp.float32),
        ],
    )
    return segsum, degree


def _segsum(tp, src, dst):
    return _sc_kernels()[0](tp, src, dst)


def _degree(dst):
    return _sc_kernels()[1](dst)


def _ln(x, g, b):
    m = jnp.mean(x, axis=-1, keepdims=True)
    v = jnp.mean((x - m) ** 2, axis=-1, keepdims=True)
    return (x - m) * lax.rsqrt(v + 1e-5) * g + b


def _mT(x, w):
    return lax.dot_general(x, w, (((1,), (1,)), ((), ())),
                           preferred_element_type=jnp.float32)


_row = pl.BlockSpec((RB, F), lambda i: (i, 0))
_par = pl.BlockSpec((1, F), lambda i: (0, 0))
_wgt = pl.BlockSpec((F, F), lambda i: (0, 0))


def _kdis_body(d0_ref, d1_ref, o_ref):
    dg = d0_ref[...] + d1_ref[...]
    o_ref[...] = jnp.where(dg > 0.0, lax.rsqrt(jnp.maximum(dg, 1.0)), 0.0)


_kdis = pl.pallas_call(
    _kdis_body,
    grid=(1,),
    in_specs=[pl.BlockSpec((80, 128), lambda i: (0, 0))] * 2,
    out_specs=pl.BlockSpec((80, 128), lambda i: (0, 0)),
    out_shape=jax.ShapeDtypeStruct((80, 128), jnp.float32),
)


def _k0_body(x_ref, win_ref, bin_ref, g_ref, b_ref, epi_ref, w0_ref, dis_ref,
             h_ref, t0_ref):
    h = _mT(x_ref[...], win_ref[...]) + bin_ref[...]
    h = _ln(h, g_ref[...], b_ref[...])
    h = jnp.maximum(h, 0.0) * epi_ref[...]
    h_ref[...] = h
    t0_ref[...] = _mT(h, w0_ref[...]) * dis_ref[...]


_k0 = pl.pallas_call(
    _k0_body,
    grid=(NBLK,),
    in_specs=[_row, _wgt, _par, _par, _par, _par, _wgt, _row],
    out_specs=[_row, _row],
    out_shape=[jax.ShapeDtypeStruct((NP, F), jnp.float32)] * 2,
)


def _k2_body(s0_ref, s1_ref, dis_ref, b_ref, g_ref, be_ref, w1_ref,
             cur_ref, t1_ref):
    dis = dis_ref[...]
    agg = dis * (s0_ref[...] + s1_ref[...]) + b_ref[...]
    cur = _ln(agg, g_ref[...], be_ref[...])
    cur_ref[...] = cur
    t1_ref[...] = _mT(cur, w1_ref[...]) * dis


_k2 = pl.pallas_call(
    _k2_body,
    grid=(NBLK,),
    in_specs=[_row, _row, _row, _par, _par, _par, _wgt],
    out_specs=[_row, _row],
    out_shape=[jax.ShapeDtypeStruct((NP, F), jnp.float32)] * 2,
)


def _k3_body(s0_ref, s1_ref, dis_ref, b_ref, g_ref, be_ref, cur_ref,
             gwa_ref, gwb_ref, gb_ref, wn_ref, curo_ref, tn_ref):
    dis = dis_ref[...]
    cur = cur_ref[...]
    h_new = _ln(dis * (s0_ref[...] + s1_ref[...]) + b_ref[...],
                g_ref[...], be_ref[...])
    gate = jax.nn.sigmoid(_mT(cur, gwa_ref[...]) + _mT(h_new, gwb_ref[...])
                          + gb_ref[...])
    curo = gate * h_new + (1.0 - gate) * cur
    curo_ref[...] = curo
    tn_ref[...] = _mT(curo, wn_ref[...]) * dis


_k3 = pl.pallas_call(
    _k3_body,
    grid=(NBLK,),
    in_specs=[_row, _row, _row, _par, _par, _par, _row, _wgt, _wgt, _par,
              _wgt],
    out_specs=[_row, _row],
    out_shape=[jax.ShapeDtypeStruct((NP, F), jnp.float32)] * 2,
)


def _k4_body(s0_ref, s1_ref, dis_ref, b_ref, g_ref, be_ref, cur_ref,
             gwa_ref, gwb_ref, gb_ref, hh_ref, rw_ref, h_ref, cn_ref,
             w0_ref, k_ref, hhn_ref, t0n_ref):
    dis = dis_ref[...]
    cur = cur_ref[...]
    h_new = _ln(dis * (s0_ref[...] + s1_ref[...]) + b_ref[...],
                g_ref[...], be_ref[...])
    gate = jax.nn.sigmoid(_mT(cur, gwa_ref[...]) + _mT(h_new, gwb_ref[...])
                          + gb_ref[...])
    curo = gate * h_new + (1.0 - gate) * cur
    kk = jnp.tanh(curo) + rw_ref[...] * hh_ref[...]
    k_ref[...] = kk
    hhn = h_ref[...] + cn_ref[...] * kk
    hhn_ref[...] = hhn
    t0n_ref[...] = _mT(hhn, w0_ref[...]) * dis


_k4 = pl.pallas_call(
    _k4_body,
    grid=(NBLK,),
    in_specs=[_row, _row, _row, _par, _par, _par, _row, _wgt, _wgt, _par,
              _row, _par, _row, _par, _wgt],
    out_specs=[_row, _row, _row],
    out_shape=[jax.ShapeDtypeStruct((NP, F), jnp.float32)] * 3,
)


def _k4f_body(s0_ref, s1_ref, dis_ref, b_ref, g_ref, be_ref, cur_ref,
              gwa_ref, gwb_ref, gb_ref, hh_ref, rw_ref, k_ref):
    dis = dis_ref[...]
    cur = cur_ref[...]
    h_new = _ln(dis * (s0_ref[...] + s1_ref[...]) + b_ref[...],
                g_ref[...], be_ref[...])
    gate = jax.nn.sigmoid(_mT(cur, gwa_ref[...]) + _mT(h_new, gwb_ref[...])
                          + gb_ref[...])
    curo = gate * h_new + (1.0 - gate) * cur
    k_ref[...] = jnp.tanh(curo) + rw_ref[...] * hh_ref[...]


_k4f = pl.pallas_call(
    _k4f_body,
    grid=(NBLK,),
    in_specs=[_row, _row, _row, _par, _par, _par, _row, _wgt, _wgt, _par,
              _row, _par],
    out_specs=_row,
    out_shape=jax.ShapeDtypeStruct((NP, F), jnp.float32),
)


def _kfin_body(h_ref, k1_ref, k2_ref, k3_ref, k4_ref, dt6_ref, wo_ref,
               bo_ref, go_ref, beo_ref, o_ref):
    i = pl.program_id(0)
    h2 = h_ref[...] + dt6_ref[...] * (k1_ref[...] + 2.0 * k2_ref[...]
                                      + 2.0 * k3_ref[...] + k4_ref[...])
    o = _ln(_mT(h2, wo_ref[...]) + bo_ref[...], go_ref[...], beo_ref[...])
    rows = lax.broadcasted_iota(jnp.int32, (RB, 1), 0) + i * RB
    o = jnp.where(rows < N, o, 0.0)
    part = jnp.sum(o, axis=0, keepdims=True) * (1.0 / N)

    @pl.when(i == 0)
    def _():
        o_ref[...] = jnp.zeros_like(o_ref)

    o_ref[...] += part


_kfin = pl.pallas_call(
    _kfin_body,
    grid=(NBLK,),
    in_specs=[_row, _row, _row, _row, _row, _par, _wgt, _par, _par, _par],
    out_specs=pl.BlockSpec((1, F), lambda i: (0, 0)),
    out_shape=jax.ShapeDtypeStruct((1, F), jnp.float32),
)


def kernel(x, edge_index, W_in, b_in, ln_in_g, ln_in_b, methylation, histones,
           log_depth, gcn_W, gcn_b, ln_g, ln_b, gate_W, gate_b,
           residual_weight, W_out, b_out, ln_out_g, ln_out_b):
    f32 = jnp.float32
    xp = jnp.pad(x, ((0, NP - N), (0, 0)))
    src = jnp.pad(edge_index[0], (0, EPAD - E),
                  constant_values=NP - 1).reshape(NW, KCH, CHUNK)
    dst = jnp.pad(edge_index[1], (0, EPAD - E),
                  constant_values=NP - 1).reshape(NW, KCH, CHUNK)

    deg2 = _degree(dst)
    d0 = deg2[0, :, 0].reshape(80, 128)
    d1 = deg2[1, :, 0].reshape(80, 128)
    dis2d = _kdis(d0, d1)
    dis_bc = jnp.broadcast_to(dis2d.reshape(NP, 1), (NP, F))

    def b2(v):
        return v.reshape(1, F)

    def full(v):
        return jnp.full((1, F), v, dtype=f32)

    ms = jnp.mean(jax.nn.sigmoid(methylation))
    hm = jax.nn.sigmoid(histones)
    activation_marks = (hm[0] + hm[2]) / 2.0
    repression_marks = (hm[1] + hm[3]) / 2.0
    acc_s = jnp.clip(activation_marks - repression_marks + 0.5, 0.0, 1.0)
    epi = acc_s * (1.0 - ms)
    depth = lax.stop_gradient(jnp.clip(jnp.exp(log_depth), MIN_DEPTH,
                                       MAX_DEPTH))
    dt = depth

    gwa = gate_W[:, :F]
    gwb = gate_W[:, F:]
    gb = b2(gate_b)
    rw = full(residual_weight)

    h, t = _k0(xp, W_in, b2(b_in), b2(ln_in_g), b2(ln_in_b), full(epi),
               gcn_W[0], dis_bc)

    def seg(tp):
        S = _segsum(tp, src, dst)
        return S[0], S[1]

    hh = h
    ks = []
    for it in range(4):
        s0, s1 = seg(t)
        cur1, t1 = _k2(s0, s1, dis_bc, b2(gcn_b[0]), b2(ln_g[0]),
                       b2(ln_b[0]), gcn_W[1])
        s0, s1 = seg(t1)
        cur2, t2 = _k3(s0, s1, dis_bc, b2(gcn_b[1]), b2(ln_g[1]),
                       b2(ln_b[1]), cur1, gwa, gwb, gb, gcn_W[2])
        s0, s1 = seg(t2)
        if it < 3:
            cnext = dt * (0.5 if it < 2 else 1.0)
            kk, hh, t = _k4(s0, s1, dis_bc, b2(gcn_b[2]), b2(ln_g[2]),
                            b2(ln_b[2]), cur2, gwa, gwb, gb, hh, rw, h,
                            full(cnext), gcn_W[0])
        else:
            kk = _k4f(s0, s1, dis_bc, b2(gcn_b[2]), b2(ln_g[2]),
                      b2(ln_b[2]), cur2, gwa, gwb, gb, hh, rw)
        ks.append(kk)

    out = _kfin(h, ks[0], ks[1], ks[2], ks[3], full(dt / 6.0), W_out,
                b2(b_out), b2(ln_out_g), b2(ln_out_b))
    return out
